# baseline (speedup 1.0000x reference)
import functools

import jax
import jax.numpy as jnp
from jax import lax
from jax.experimental import pallas as pl
from jax.experimental.pallas import tpu as pltpu
from jax.experimental.pallas import tpu_sc as plsc

N_CODES = 8192
DIM = 32
N_ROWS = 8192
R_BLK = 1024
C_BLK = 2048
N_RB = N_ROWS // R_BLK
N_CB = N_CODES // C_BLK


def _rne_bf16(v):
    b = lax.bitcast_convert_type(v, jnp.int32)
    r = (b + 0x7FFF + ((b >> 16) & 1)) & jnp.int32(-65536)
    return lax.bitcast_convert_type(r, jnp.float32)


def _dist_body(xb_ref, e_ref, x2_ref, e2_ref, idx_ref, minv, wtile, wloc):
    c = pl.program_id(1)
    xb = xb_ref[...]
    e = e_ref[...]
    mm = lax.dot_general(xb, e, (((1,), (1,)), ((), ())),
                         preferred_element_type=jnp.float32)
    d2 = (x2_ref[...][:, None] - mm) + e2_ref[...][None, :]
    m = jnp.min(d2, axis=1)
    s = jnp.sqrt(jnp.maximum(m, 0.0))
    bits0 = lax.bitcast_convert_type(s * s, jnp.int32)
    big = jnp.float32(-jnp.inf)
    bound = jnp.full_like(m, big)
    for k in range(-3, 5):
        xk = lax.bitcast_convert_type(bits0 + k, jnp.float32)
        ok = jnp.sqrt(xk) == s
        bound = jnp.maximum(bound, jnp.where(ok, xk, big))
    bound = jnp.maximum(bound, m)
    col = lax.broadcasted_iota(jnp.int32, (R_BLK, C_BLK), 1)
    loc = jnp.min(jnp.where(d2 <= bound[:, None], col, C_BLK), axis=1)

    @pl.when(c == 0)
    def _():
        minv[...] = _rne_bf16(s)
        wtile[...] = jnp.zeros((R_BLK,), jnp.int32)
        wloc[...] = loc

    @pl.when(c > 0)
    def _():
        better = s < minv[...]
        minv[...] = jnp.where(better, _rne_bf16(s), minv[...])
        wtile[...] = jnp.where(better, c, wtile[...])
        wloc[...] = jnp.where(better, loc, wloc[...])

    @pl.when(c == N_CB - 1)
    def _():
        idx_ref[...] = wtile[...] * C_BLK + wloc[...]


_dist_call = pl.pallas_call(
    _dist_body,
    grid=(N_RB, N_CB),
    in_specs=[
        pl.BlockSpec((R_BLK, DIM), lambda r, c: (r, 0)),
        pl.BlockSpec((C_BLK, DIM), lambda r, c: (c, 0)),
        pl.BlockSpec((R_BLK,), lambda r, c: (r,)),
        pl.BlockSpec((C_BLK,), lambda r, c: (c,)),
    ],
    out_specs=pl.BlockSpec((R_BLK,), lambda r, c: (r,)),
    out_shape=jax.ShapeDtypeStruct((N_ROWS,), jnp.int32),
    scratch_shapes=[
        pltpu.VMEM((R_BLK,), jnp.float32),
        pltpu.VMEM((R_BLK,), jnp.int32),
        pltpu.VMEM((R_BLK,), jnp.int32),
    ],
    compiler_params=pltpu.CompilerParams(
        dimension_semantics=("arbitrary", "arbitrary")),
)


_NC = 2
_NS = 16
_NW = _NC * _NS
_BPW = N_ROWS // _NW
_HALF = _BPW // 2

@functools.cache
def _sc_gather_hist_call():
    mesh = plsc.VectorSubcoreMesh(core_axis_name="c", subcore_axis_name="s")
    return pl.kernel(
        _sc_gather_hist,
        out_type=[
            jax.ShapeDtypeStruct((N_ROWS, DIM), jnp.float32),
            jax.ShapeDtypeStruct((_NC, N_CODES), jnp.float32),
        ],
        mesh=mesh,
        scratch_types=[
            pltpu.VMEM((_HALF,), jnp.int32),
            pltpu.VMEM((_HALF,), jnp.int32),
            pltpu.VMEM((_BPW, DIM), jnp.float32),
            pltpu.VMEM((_HALF,), jnp.float32),
            pltpu.VMEM_SHARED((N_CODES,), jnp.float32),
            pltpu.SemaphoreType.DMA,
        ],
        compiler_params=pltpu.CompilerParams(use_tc_tiling_on_sc=False),
    )


def _sc_gather_hist(table_hbm, idx_hbm, zeros_hbm, quant_hbm, counts_hbm,
                    idx_a, idx_b, rows_v, ones_v, hist_sh, sem):
    cid = lax.axis_index("c")
    sid = lax.axis_index("s")
    wid = sid * _NC + cid
    base = wid * _BPW

    @pl.when(sid == 0)
    def _():
        pltpu.sync_copy(zeros_hbm.at[cid], hist_sh)

    pltpu.sync_copy(idx_hbm.at[pl.ds(base, _HALF)], idx_a)
    pltpu.sync_copy(idx_hbm.at[pl.ds(base + _HALF, _HALF)], idx_b)

    cp0 = pltpu.async_copy(table_hbm.at[idx_a], rows_v.at[pl.ds(0, _HALF)], sem)
    cp1 = pltpu.async_copy(table_hbm.at[idx_b], rows_v.at[pl.ds(_HALF, _HALF)],
                           sem)
    cp0.wait()
    cp1.wait()
    pltpu.sync_copy(rows_v, quant_hbm.at[pl.ds(base, _BPW)])

    for i in range(_HALF // 16):
        ones_v[pl.ds(i * 16, 16)] = jnp.ones((16,), jnp.float32)
    plsc.subcore_barrier()
    pltpu.sync_copy(ones_v, hist_sh.at[idx_a], add=True)
    pltpu.sync_copy(ones_v, hist_sh.at[idx_b], add=True)
    plsc.subcore_barrier()

    @pl.when(sid == 0)
    def _():
        pltpu.sync_copy(hist_sh, counts_hbm.at[cid])


def _scalar_body(x_ref, q_ref, pc_ref, w_ref, loss_ref, perp_ref, usage_ref):
    x = x_ref[...]
    q = q_ref[...]
    diff = q - x
    loss = jnp.sum(diff * diff) * (1.0 / (N_ROWS * DIM))
    loss_ref[...] = jnp.reshape(loss, (1, 1))
    counts = pc_ref[0, :] + pc_ref[1, :]
    p = counts * (1.0 / N_ROWS)
    ent = jnp.sum(p * jnp.log(p + 1e-10))
    perp_ref[...] = jnp.reshape(jnp.exp(-ent), (1, 1))
    usage = jnp.sum((w_ref[...] >= 0.01).astype(jnp.int32))
    usage_ref[...] = jnp.reshape(usage, (1, 1))


_scalar_call = pl.pallas_call(
    _scalar_body,
    out_shape=[
        jax.ShapeDtypeStruct((1, 1), jnp.float32),
        jax.ShapeDtypeStruct((1, 1), jnp.float32),
        jax.ShapeDtypeStruct((1, 1), jnp.int32),
    ],
)


def kernel(inputs, embedding_weight, weight):
    x_flat = jnp.transpose(inputs, (0, 2, 3, 1)).reshape(N_ROWS, DIM)
    xb = (2.0 * x_flat).astype(jnp.bfloat16).astype(jnp.float32)
    x2 = jnp.sum(x_flat * x_flat, axis=1)
    e2 = jnp.sum(embedding_weight * embedding_weight, axis=1)
    indices = _dist_call(xb, embedding_weight, x2, e2)
    zeros = jnp.zeros((_NC, N_CODES), jnp.float32)
    quant, pcounts = _sc_gather_hist_call()(embedding_weight, indices, zeros)
    loss, perp, usage = _scalar_call(x_flat, quant, pcounts, weight)
    out = jnp.transpose(quant.reshape(inputs.shape[0], 32, 32, DIM),
                        (0, 3, 1, 2))
    return (out, loss.reshape(()), perp.reshape(()), usage.reshape(()),
            indices)


# speedup vs baseline: 1.2345x; 1.2345x over previous
import functools

import jax
import jax.numpy as jnp
from jax import lax
from jax.experimental import pallas as pl
from jax.experimental.pallas import tpu as pltpu
from jax.experimental.pallas import tpu_sc as plsc

N_CODES = 8192
DIM = 32
N_ROWS = 8192
R_BLK = 1024
C_BLK = 2048
N_RB = N_ROWS // R_BLK
N_CB = N_CODES // C_BLK


def _rne_bf16(v):
    b = lax.bitcast_convert_type(v, jnp.int32)
    r = (b + 0x7FFF + ((b >> 16) & 1)) & jnp.int32(-65536)
    return lax.bitcast_convert_type(r, jnp.float32)


def _dist_body(xb_ref, e_ref, x2_ref, e2_ref, idx_ref, minv, wtile, wloc,
               mrow):
    c = pl.program_id(1)
    xb = xb_ref[...]
    e = e_ref[...]
    mm = lax.dot_general(xb, e, (((1,), (1,)), ((), ())),
                         preferred_element_type=jnp.float32)
    d2 = (x2_ref[...][:, None] - mm) + e2_ref[...][None, :]
    mrow[...] = jnp.min(d2, axis=1)
    m = mrow[...]
    s = jnp.sqrt(jnp.maximum(m, 0.0))
    bits0 = lax.bitcast_convert_type(s * s, jnp.int32)
    big = jnp.float32(-jnp.inf)
    bound = jnp.full_like(m, big)
    for k in range(-3, 5):
        xk = lax.bitcast_convert_type(bits0 + k, jnp.float32)
        ok = jnp.sqrt(xk) == s
        bound = jnp.maximum(bound, jnp.where(ok, xk, big))
    bound = jnp.maximum(bound, m)
    col = lax.broadcasted_iota(jnp.int32, (R_BLK, C_BLK), 1)
    loc = jnp.min(jnp.where(d2 <= bound[:, None], col, C_BLK), axis=1)

    @pl.when(c == 0)
    def _():
        minv[...] = _rne_bf16(s)
        wtile[...] = jnp.zeros((R_BLK,), jnp.int32)
        wloc[...] = loc

    @pl.when(c > 0)
    def _():
        better = s < minv[...]
        minv[...] = jnp.where(better, _rne_bf16(s), minv[...])
        wtile[...] = jnp.where(better, c, wtile[...])
        wloc[...] = jnp.where(better, loc, wloc[...])

    @pl.when(c == N_CB - 1)
    def _():
        idx_ref[...] = wtile[...] * C_BLK + wloc[...]


_dist_call = pl.pallas_call(
    _dist_body,
    grid=(N_RB, N_CB),
    in_specs=[
        pl.BlockSpec((R_BLK, DIM), lambda r, c: (r, 0)),
        pl.BlockSpec((C_BLK, DIM), lambda r, c: (c, 0)),
        pl.BlockSpec((R_BLK,), lambda r, c: (r,)),
        pl.BlockSpec((C_BLK,), lambda r, c: (c,)),
    ],
    out_specs=pl.BlockSpec((R_BLK,), lambda r, c: (r,)),
    out_shape=jax.ShapeDtypeStruct((N_ROWS,), jnp.int32),
    scratch_shapes=[
        pltpu.VMEM((R_BLK,), jnp.float32),
        pltpu.VMEM((R_BLK,), jnp.int32),
        pltpu.VMEM((R_BLK,), jnp.int32),
        pltpu.VMEM((R_BLK,), jnp.float32),
    ],
    compiler_params=pltpu.CompilerParams(
        dimension_semantics=("arbitrary", "arbitrary")),
)


_NC = 2
_NS = 16
_NW = _NC * _NS
_BPW = N_ROWS // _NW
_HALF = _BPW // 2

@functools.cache
def _sc_gather_hist_call():
    mesh = plsc.VectorSubcoreMesh(core_axis_name="c", subcore_axis_name="s")
    return pl.kernel(
        _sc_gather_hist,
        out_type=[
            jax.ShapeDtypeStruct((N_ROWS, DIM), jnp.float32),
            jax.ShapeDtypeStruct((_NC, N_CODES), jnp.float32),
        ],
        mesh=mesh,
        scratch_types=[
            pltpu.VMEM((_HALF,), jnp.int32),
            pltpu.VMEM((_HALF,), jnp.int32),
            pltpu.VMEM((_BPW, DIM), jnp.float32),
            pltpu.VMEM((_HALF,), jnp.float32),
            pltpu.VMEM_SHARED((N_CODES,), jnp.float32),
            pltpu.SemaphoreType.DMA,
        ],
        compiler_params=pltpu.CompilerParams(use_tc_tiling_on_sc=False),
    )


def _sc_gather_hist(table_hbm, idx_hbm, zeros_hbm, quant_hbm, counts_hbm,
                    idx_a, idx_b, rows_v, ones_v, hist_sh, sem):
    cid = lax.axis_index("c")
    sid = lax.axis_index("s")
    wid = sid * _NC + cid
    base = wid * _BPW

    @pl.when(sid == 0)
    def _():
        pltpu.sync_copy(zeros_hbm.at[cid], hist_sh)

    pltpu.sync_copy(idx_hbm.at[pl.ds(base, _HALF)], idx_a)
    pltpu.sync_copy(idx_hbm.at[pl.ds(base + _HALF, _HALF)], idx_b)

    cp0 = pltpu.async_copy(table_hbm.at[idx_a], rows_v.at[pl.ds(0, _HALF)], sem)
    cp1 = pltpu.async_copy(table_hbm.at[idx_b], rows_v.at[pl.ds(_HALF, _HALF)],
                           sem)
    cp0.wait()
    cp1.wait()
    pltpu.sync_copy(rows_v, quant_hbm.at[pl.ds(base, _BPW)])

    for i in range(_HALF // 16):
        ones_v[pl.ds(i * 16, 16)] = jnp.ones((16,), jnp.float32)
    plsc.subcore_barrier()
    pltpu.sync_copy(ones_v, hist_sh.at[idx_a], add=True)
    pltpu.sync_copy(ones_v, hist_sh.at[idx_b], add=True)
    plsc.subcore_barrier()

    @pl.when(sid == 0)
    def _():
        pltpu.sync_copy(hist_sh, counts_hbm.at[cid])


def _scalar_body(x_ref, q_ref, pc_ref, w_ref, loss_ref, perp_ref, usage_ref):
    x = x_ref[...]
    q = q_ref[...]
    diff = q - x
    loss = jnp.sum(diff * diff) * (1.0 / (N_ROWS * DIM))
    loss_ref[...] = jnp.reshape(loss, (1, 1))
    counts = pc_ref[0, :] + pc_ref[1, :]
    p = counts * (1.0 / N_ROWS)
    ent = jnp.sum(p * jnp.log(p + 1e-10))
    perp_ref[...] = jnp.reshape(jnp.exp(-ent), (1, 1))
    usage = jnp.sum((w_ref[...] >= 0.01).astype(jnp.int32))
    usage_ref[...] = jnp.reshape(usage, (1, 1))


_scalar_call = pl.pallas_call(
    _scalar_body,
    out_shape=[
        jax.ShapeDtypeStruct((1, 1), jnp.float32),
        jax.ShapeDtypeStruct((1, 1), jnp.float32),
        jax.ShapeDtypeStruct((1, 1), jnp.int32),
    ],
)


def kernel(inputs, embedding_weight, weight):
    x_flat = jnp.transpose(inputs, (0, 2, 3, 1)).reshape(N_ROWS, DIM)
    xb = (2.0 * x_flat).astype(jnp.bfloat16).astype(jnp.float32)
    x2 = jnp.sum(x_flat * x_flat, axis=1)
    e2 = jnp.sum(embedding_weight * embedding_weight, axis=1)
    indices = _dist_call(xb, embedding_weight, x2, e2)
    zeros = jnp.zeros((_NC, N_CODES), jnp.float32)
    quant, pcounts = _sc_gather_hist_call()(embedding_weight, indices, zeros)
    loss, perp, usage = _scalar_call(x_flat, quant, pcounts, weight)
    out = jnp.transpose(quant.reshape(inputs.shape[0], 32, 32, DIM),
                        (0, 3, 1, 2))
    return (out, loss.reshape(()), perp.reshape(()), usage.reshape(()),
            indices)
